# Initial kernel scaffold; baseline (speedup 1.0000x reference)
import jax
import jax.numpy as jnp
from jax.experimental import pallas as pl


def kernel(labels, emb_table, W, b):
    raise NotImplementedError("write your pallas kernel here")


# speedup vs baseline: 20.3111x; 20.3111x over previous
import functools

import jax
import jax.numpy as jnp
from jax import lax
from jax.experimental import pallas as pl
from jax.experimental.pallas import tpu as pltpu
from jax.experimental.pallas import tpu_sc as plsc

VOCAB = 100000
EMB = 64
B = 4096
L = 200
EPS = 1e-06
NEG = -1.0 / EPS

NC = 2
NS = 16
NW = NC * NS
ROWS_PER_TILE = B // NW
RC = 32
NCHUNK = ROWS_PER_TILE // RC
CHUNK_WORDS = RC * L
LAB_BUF = CHUNK_WORDS + 16
NFULL = L // 16
REM = L - NFULL * 16


_BV = 5000


def _matvec_body(emb_ref, w_ref, t_ref):
    t_ref[...] = jnp.sum(emb_ref[...] * w_ref[...], axis=1, keepdims=True)


def _matvec(emb_table, w_row):
    return pl.pallas_call(
        _matvec_body,
        grid=(VOCAB // _BV,),
        in_specs=[
            pl.BlockSpec((_BV, EMB), lambda i: (i, 0)),
            pl.BlockSpec((1, EMB), lambda i: (0, 0)),
        ],
        out_specs=pl.BlockSpec((_BV, 1), lambda i: (i, 0)),
        out_shape=jax.ShapeDtypeStruct((VOCAB, 1), jnp.float32),
    )(emb_table, w_row)


_mesh = plsc.VectorSubcoreMesh(core_axis_name="c", subcore_axis_name="s")


@functools.partial(
    pl.kernel,
    mesh=_mesh,
    compiler_params=pltpu.CompilerParams(needs_layout_passes=False),
    out_type=[
        jax.ShapeDtypeStruct((B * 16,), jnp.float32),
        jax.ShapeDtypeStruct((B * 16,), jnp.float32),
    ],
    scratch_types=[
        pltpu.VMEM((VOCAB,), jnp.float32),
        pltpu.VMEM((LAB_BUF,), jnp.int32),
        pltpu.VMEM((ROWS_PER_TILE * 16,), jnp.float32),
        pltpu.VMEM((ROWS_PER_TILE * 16,), jnp.float32),
    ],
)
def _pool(t_hbm, lab_hbm, s_hbm, c_hbm, t_v, lab_v, s_v, c_v):
    wid = lax.axis_index("s") * NC + lax.axis_index("c")
    pltpu.sync_copy(t_hbm, t_v)
    lane = lax.broadcasted_iota(jnp.int32, (16,), 0)
    rem_mask = lane < REM
    for chunk in range(NCHUNK):
        src = wid * (ROWS_PER_TILE * L) + chunk * CHUNK_WORDS
        pltpu.sync_copy(
            lab_hbm.at[pl.ds(src, CHUNK_WORDS)],
            lab_v.at[pl.ds(0, CHUNK_WORDS)],
        )

        def row_body(r, carry, chunk=chunk):
            base = r * L
            acc_s = jnp.zeros((16,), jnp.float32)
            acc_c = jnp.zeros((16,), jnp.float32)
            for g16 in range(NFULL + 1):
                lab = lab_v[pl.ds(base + g16 * 16, 16)]
                msk = lab > 0
                if g16 == NFULL:
                    msk = msk & rem_mask
                idx = jnp.where(msk, lab, 0)
                vals = plsc.load_gather(t_v, [idx])
                acc_s = acc_s + jnp.where(msk, vals, 0.0)
                acc_c = acc_c + jnp.where(msk, 1.0, 0.0)
            off = (chunk * RC + r) * 16
            s_v[pl.ds(off, 16)] = acc_s
            c_v[pl.ds(off, 16)] = acc_c
            return carry

        lax.fori_loop(0, RC, row_body, 0)
    out = wid * (ROWS_PER_TILE * 16)
    pltpu.sync_copy(s_v, s_hbm.at[pl.ds(out, ROWS_PER_TILE * 16)])
    pltpu.sync_copy(c_v, c_hbm.at[pl.ds(out, ROWS_PER_TILE * 16)])


_BR = 256


def _softmax_body(s2_ref, c2_ref, b_ref, lab_ref, y_ref, a_ref, sm_ref):
    @pl.when(pl.program_id(0) == 0)
    def _():
        s = jnp.sum(s2_ref[...], axis=1, keepdims=True)
        c = jnp.sum(c2_ref[...], axis=1, keepdims=True)
        x = s / (EPS + c) + b_ref[0, 0]
        xm = jnp.where(c > 0, x, NEG)
        n_inv = B * L - jnp.sum(c)
        mrow = jnp.max(xm)
        m = jnp.where(n_inv > 0, jnp.maximum(mrow, NEG), mrow)
        e = jnp.exp(xm - m)
        e_neg = jnp.exp(NEG - m)
        denom = jnp.sum(c * e) + n_inv * e_neg
        a_ref[...] = e / denom
        sm_ref[0] = e_neg / denom

    i = pl.program_id(0)
    a_blk = a_ref[pl.ds(i * _BR, _BR), :]
    y_ref[...] = jnp.where(lab_ref[...] > 0, a_blk, sm_ref[0])


def _softmax(s2, c2, b11, labels):
    return pl.pallas_call(
        _softmax_body,
        grid=(B // _BR,),
        in_specs=[
            pl.BlockSpec((B, 16), lambda i: (0, 0)),
            pl.BlockSpec((B, 16), lambda i: (0, 0)),
            pl.BlockSpec((1, 1), lambda i: (0, 0)),
            pl.BlockSpec((_BR, L), lambda i: (i, 0)),
        ],
        out_specs=pl.BlockSpec((_BR, L), lambda i: (i, 0)),
        out_shape=jax.ShapeDtypeStruct((B, L), jnp.float32),
        scratch_shapes=[
            pltpu.VMEM((B, 1), jnp.float32),
            pltpu.SMEM((1,), jnp.float32),
        ],
    )(s2, c2, b11, labels)


def kernel(labels, emb_table, W, b):
    t = _matvec(emb_table, W.reshape(1, EMB)).reshape(VOCAB)
    s_flat, c_flat = _pool(t, labels.reshape(B * L))
    s2 = s_flat.reshape(B, 16)
    c2 = c_flat.reshape(B, 16)
    return _softmax(s2, c2, b.reshape(1, 1), labels)
